# baseline (speedup 1.0000x reference)
import jax
import jax.numpy as jnp
from jax import lax
from jax.experimental import pallas as pl
from jax.experimental.pallas import tpu as pltpu
from jax.experimental.pallas import tpu_sc as plsc

L = 16
NC = 2
NS = 16
NW = NC * NS
B = 16384
D = 64
W = 2 * D
BPW = B // NW
CHUNK = 128
NCHUNK = BPW // CHUNK
NBUF = 2


def _body(uid_hbm, iid_hbm, jid_hbm, p_hbm, q_hbm,
          predi_hbm, predj_hbm,
          uid_v, iid_v, jid_v, ru_v, ri_v, rj_v,
          u_rows, qi_rows, qj_rows,
          out_i, out_j, sems):
    cid = lax.axis_index("c")
    sid = lax.axis_index("s")
    wid = sid * NC + cid

    pltpu.sync_copy(uid_hbm.at[wid], uid_v)
    pltpu.sync_copy(iid_hbm.at[wid], iid_v)
    pltpu.sync_copy(jid_hbm.at[wid], jid_v)

    def shift_chunk(kc, _):
        def shift_vec(v, _):
            s = pl.ds(v * L, L)
            u = uid_v[kc, s]
            i = iid_v[kc, s]
            j = jid_v[kc, s]
            ru_v[kc, s] = jnp.where(u >= HALF, u - HALF, u)
            ri_v[kc, s] = jnp.where(i >= HALF, i - HALF, i)
            rj_v[kc, s] = jnp.where(j >= HALF, j - HALF, j)
            return 0
        lax.fori_loop(0, CHUNK // L, shift_vec, 0, unroll=4)
        return 0

    lax.fori_loop(0, NCHUNK, shift_chunk, 0)

    def fire(k):
        buf = k % NBUF
        cps = (
            pltpu.async_copy(p_hbm.at[ru_v.at[k]], u_rows.at[buf], sems.at[k]),
            pltpu.async_copy(q_hbm.at[ri_v.at[k]], qi_rows.at[buf], sems.at[k]),
            pltpu.async_copy(q_hbm.at[rj_v.at[k]], qj_rows.at[buf], sems.at[k]),
        )
        return cps

    inflight = [fire(0), fire(1)]

    lane = lax.iota(jnp.int32, L)

    def group(g, carry):
        k, buf = carry
        base = g * L
        pu_vec = jnp.where(uid_v[k, pl.ds(base, L)] >= HALF, D, 0)
        pi_vec = jnp.where(iid_v[k, pl.ds(base, L)] >= HALF, D, 0)
        pj_vec = jnp.where(jid_v[k, pl.ds(base, L)] >= HALF, D, 0)
        res_i = jnp.zeros((L,), jnp.float32)
        res_j = jnp.zeros((L,), jnp.float32)
        for e2 in range(L):
            e = base + e2
            pu = pu_vec[e2]
            pi = pi_vec[e2]
            pj = pj_vec[e2]
            ai = jnp.zeros((L,), jnp.float32)
            aj = jnp.zeros((L,), jnp.float32)
            for c in range(D // L):
                u = u_rows[buf, e, pl.ds(pu + c * L, L)]
                qi = qi_rows[buf, e, pl.ds(pi + c * L, L)]
                qj = qj_rows[buf, e, pl.ds(pj + c * L, L)]
                ai = ai + u * qi
                aj = aj + u * qj
            here = lane == e2
            res_i = jnp.where(here, jnp.broadcast_to(jnp.sum(ai), (L,)), res_i)
            res_j = jnp.where(here, jnp.broadcast_to(jnp.sum(aj), (L,)), res_j)
        out_i[pl.ds(k * CHUNK + base, L)] = res_i
        out_j[pl.ds(k * CHUNK + base, L)] = res_j
        return carry

    for k in range(NCHUNK):
        for cp in inflight[k % NBUF]:
            cp.wait()
        lax.fori_loop(0, CHUNK // L, group, (k, k % NBUF))
        if k + NBUF < NCHUNK:
            inflight[k % NBUF] = fire(k + NBUF)

    pltpu.sync_copy(out_i, predi_hbm.at[pl.ds(wid * BPW, BPW)])
    pltpu.sync_copy(out_j, predj_hbm.at[pl.ds(wid * BPW, BPW)])


HALF = 50048
_TBW = 17 * 128


def _repack_body(a_ref, b_ref, out_ref):
    left = jnp.swapaxes(a_ref[...], 0, 1)
    right = jnp.swapaxes(b_ref[...], 0, 1)
    out_ref[...] = jnp.concatenate([left, right], axis=1)


def _repack(table):
    xt = jnp.swapaxes(table, 0, 1)
    return pl.pallas_call(
        _repack_body,
        grid=(HALF // _TBW,),
        in_specs=[
            pl.BlockSpec((D, _TBW), lambda i: (0, i)),
            pl.BlockSpec((D, _TBW), lambda i: (0, HALF // _TBW + i)),
        ],
        out_specs=pl.BlockSpec((_TBW, W), lambda i: (i, 0)),
        out_shape=jax.ShapeDtypeStruct((HALF, W), jnp.float32),
    )(xt, xt)


def kernel(uid, iid, jid, P, Q):
    uid3 = uid.reshape(NW, NCHUNK, CHUNK)
    iid3 = iid.reshape(NW, NCHUNK, CHUNK)
    jid3 = jid.reshape(NW, NCHUNK, CHUNK)
    p2 = _repack(P)
    q2 = _repack(Q)
    assert p2.shape == (HALF, W) and q2.shape == (HALF, W)

    mesh = plsc.VectorSubcoreMesh(core_axis_name="c", subcore_axis_name="s")
    fn = pl.kernel(
        _body,
        out_type=(
            jax.ShapeDtypeStruct((B,), jnp.float32),
            jax.ShapeDtypeStruct((B,), jnp.float32),
        ),
        mesh=mesh,
        compiler_params=pltpu.CompilerParams(
            needs_layout_passes=False, use_tc_tiling_on_sc=True
        ),
        scratch_types=[
            pltpu.VMEM((NCHUNK, CHUNK), jnp.int32),
            pltpu.VMEM((NCHUNK, CHUNK), jnp.int32),
            pltpu.VMEM((NCHUNK, CHUNK), jnp.int32),
            pltpu.VMEM((NCHUNK, CHUNK), jnp.int32),
            pltpu.VMEM((NCHUNK, CHUNK), jnp.int32),
            pltpu.VMEM((NCHUNK, CHUNK), jnp.int32),
            pltpu.VMEM((NBUF, CHUNK, W), jnp.float32),
            pltpu.VMEM((NBUF, CHUNK, W), jnp.float32),
            pltpu.VMEM((NBUF, CHUNK, W), jnp.float32),
            pltpu.VMEM((BPW,), jnp.float32),
            pltpu.VMEM((BPW,), jnp.float32),
            pltpu.SemaphoreType.DMA((NCHUNK,)),
        ],
    )
    predi, predj = fn(uid3, iid3, jid3, p2, q2)
    return predi, predj


# speedup vs baseline: 1.0170x; 1.0170x over previous
import jax
import jax.numpy as jnp
from jax import lax
from jax.experimental import pallas as pl
from jax.experimental.pallas import tpu as pltpu
from jax.experimental.pallas import tpu_sc as plsc

L = 16
NC = 2
NS = 16
NW = NC * NS
B = 16384
D = 64
W = 2 * D
BPW = B // NW
CHUNK = 64
NCHUNK = BPW // CHUNK
NBUF = 4


def _body(uid_hbm, iid_hbm, jid_hbm, p_hbm, q_hbm,
          predi_hbm, predj_hbm,
          uid_v, iid_v, jid_v, ru_v, ri_v, rj_v,
          u_rows, qi_rows, qj_rows,
          out_i, out_j, sems):
    cid = lax.axis_index("c")
    sid = lax.axis_index("s")
    wid = sid * NC + cid

    pltpu.sync_copy(uid_hbm.at[wid], uid_v)
    pltpu.sync_copy(iid_hbm.at[wid], iid_v)
    pltpu.sync_copy(jid_hbm.at[wid], jid_v)

    def shift_chunk(kc, _):
        def shift_vec(v, _):
            s = pl.ds(v * L, L)
            u = uid_v[kc, s]
            i = iid_v[kc, s]
            j = jid_v[kc, s]
            ru_v[kc, s] = jnp.where(u >= HALF, u - HALF, u)
            ri_v[kc, s] = jnp.where(i >= HALF, i - HALF, i)
            rj_v[kc, s] = jnp.where(j >= HALF, j - HALF, j)
            return 0
        lax.fori_loop(0, CHUNK // L, shift_vec, 0, unroll=4)
        return 0

    lax.fori_loop(0, NCHUNK, shift_chunk, 0)

    def fire(k):
        buf = k % NBUF
        cps = (
            pltpu.async_copy(p_hbm.at[ru_v.at[k]], u_rows.at[buf], sems.at[k]),
            pltpu.async_copy(q_hbm.at[ri_v.at[k]], qi_rows.at[buf], sems.at[k]),
            pltpu.async_copy(q_hbm.at[rj_v.at[k]], qj_rows.at[buf], sems.at[k]),
        )
        return cps

    inflight = [fire(k) for k in range(NBUF)]

    lane = lax.iota(jnp.int32, L)

    def group(g, carry):
        k, buf = carry
        base = g * L
        pu_vec = jnp.where(uid_v[k, pl.ds(base, L)] >= HALF, D, 0)
        pi_vec = jnp.where(iid_v[k, pl.ds(base, L)] >= HALF, D, 0)
        pj_vec = jnp.where(jid_v[k, pl.ds(base, L)] >= HALF, D, 0)
        res_i = jnp.zeros((L,), jnp.float32)
        res_j = jnp.zeros((L,), jnp.float32)
        for e2 in range(L):
            e = base + e2
            pu = pu_vec[e2]
            pi = pi_vec[e2]
            pj = pj_vec[e2]
            ai = jnp.zeros((L,), jnp.float32)
            aj = jnp.zeros((L,), jnp.float32)
            for c in range(D // L):
                u = u_rows[buf, e, pl.ds(pu + c * L, L)]
                qi = qi_rows[buf, e, pl.ds(pi + c * L, L)]
                qj = qj_rows[buf, e, pl.ds(pj + c * L, L)]
                ai = ai + u * qi
                aj = aj + u * qj
            here = lane == e2
            res_i = jnp.where(here, jnp.broadcast_to(jnp.sum(ai), (L,)), res_i)
            res_j = jnp.where(here, jnp.broadcast_to(jnp.sum(aj), (L,)), res_j)
        out_i[pl.ds(k * CHUNK + base, L)] = res_i
        out_j[pl.ds(k * CHUNK + base, L)] = res_j
        return carry

    for k in range(NCHUNK):
        for cp in inflight[k % NBUF]:
            cp.wait()
        lax.fori_loop(0, CHUNK // L, group, (k, k % NBUF))
        if k + NBUF < NCHUNK:
            inflight[k % NBUF] = fire(k + NBUF)

    pltpu.sync_copy(out_i, predi_hbm.at[pl.ds(wid * BPW, BPW)])
    pltpu.sync_copy(out_j, predj_hbm.at[pl.ds(wid * BPW, BPW)])


HALF = 50048
_TBW = 23 * 128


def _repack_body(a_ref, b_ref, out_ref):
    left = jnp.swapaxes(a_ref[...], 0, 1)
    right = jnp.swapaxes(b_ref[...], 0, 1)
    out_ref[...] = jnp.concatenate([left, right], axis=1)


def _repack(table):
    xt = jnp.swapaxes(table, 0, 1)
    return pl.pallas_call(
        _repack_body,
        grid=(HALF // _TBW,),
        in_specs=[
            pl.BlockSpec((D, _TBW), lambda i: (0, i)),
            pl.BlockSpec((D, _TBW), lambda i: (0, HALF // _TBW + i)),
        ],
        out_specs=pl.BlockSpec((_TBW, W), lambda i: (i, 0)),
        out_shape=jax.ShapeDtypeStruct((HALF, W), jnp.float32),
    )(xt, xt)


def kernel(uid, iid, jid, P, Q):
    uid3 = uid.reshape(NW, NCHUNK, CHUNK)
    iid3 = iid.reshape(NW, NCHUNK, CHUNK)
    jid3 = jid.reshape(NW, NCHUNK, CHUNK)
    p2 = _repack(P)
    q2 = _repack(Q)
    assert p2.shape == (HALF, W) and q2.shape == (HALF, W)

    mesh = plsc.VectorSubcoreMesh(core_axis_name="c", subcore_axis_name="s")
    fn = pl.kernel(
        _body,
        out_type=(
            jax.ShapeDtypeStruct((B,), jnp.float32),
            jax.ShapeDtypeStruct((B,), jnp.float32),
        ),
        mesh=mesh,
        compiler_params=pltpu.CompilerParams(
            needs_layout_passes=False, use_tc_tiling_on_sc=True
        ),
        scratch_types=[
            pltpu.VMEM((NCHUNK, CHUNK), jnp.int32),
            pltpu.VMEM((NCHUNK, CHUNK), jnp.int32),
            pltpu.VMEM((NCHUNK, CHUNK), jnp.int32),
            pltpu.VMEM((NCHUNK, CHUNK), jnp.int32),
            pltpu.VMEM((NCHUNK, CHUNK), jnp.int32),
            pltpu.VMEM((NCHUNK, CHUNK), jnp.int32),
            pltpu.VMEM((NBUF, CHUNK, W), jnp.float32),
            pltpu.VMEM((NBUF, CHUNK, W), jnp.float32),
            pltpu.VMEM((NBUF, CHUNK, W), jnp.float32),
            pltpu.VMEM((BPW,), jnp.float32),
            pltpu.VMEM((BPW,), jnp.float32),
            pltpu.SemaphoreType.DMA((NCHUNK,)),
        ],
    )
    predi, predj = fn(uid3, iid3, jid3, p2, q2)
    return predi, predj


# speedup vs baseline: 1.0679x; 1.0501x over previous
import jax
import jax.numpy as jnp
from jax import lax
from jax.experimental import pallas as pl
from jax.experimental.pallas import tpu as pltpu
from jax.experimental.pallas import tpu_sc as plsc

L = 16
NC = 2
NS = 16
NW = NC * NS
B = 16384
D = 64
W = 2 * D
BPW = B // NW
CHUNK = 128
NCHUNK = BPW // CHUNK
NBUF = 2


def _body(uid_hbm, iid_hbm, jid_hbm, p_hbm, q_hbm,
          predi_hbm, predj_hbm,
          uid_v, iid_v, jid_v, ru_v, ri_v, rj_v,
          u_rows, qi_rows, qj_rows,
          out_i, out_j, sems):
    cid = lax.axis_index("c")
    sid = lax.axis_index("s")
    wid = sid * NC + cid

    pltpu.sync_copy(uid_hbm.at[wid], uid_v)
    pltpu.sync_copy(iid_hbm.at[wid], iid_v)
    pltpu.sync_copy(jid_hbm.at[wid], jid_v)

    def shift_chunk(kc, _):
        def shift_vec(v, _):
            s = pl.ds(v * L, L)
            u = uid_v[kc, s]
            i = iid_v[kc, s]
            j = jid_v[kc, s]
            ru_v[kc, s] = jnp.where(u >= HALF, u - HALF, u)
            ri_v[kc, s] = jnp.where(i >= HALF, i - HALF, i)
            rj_v[kc, s] = jnp.where(j >= HALF, j - HALF, j)
            return 0
        lax.fori_loop(0, CHUNK // L, shift_vec, 0, unroll=4)
        return 0

    lax.fori_loop(0, NCHUNK, shift_chunk, 0)

    def fire(k):
        buf = k % NBUF
        cps = (
            pltpu.async_copy(p_hbm.at[ru_v.at[k]], u_rows.at[buf], sems.at[k]),
            pltpu.async_copy(q_hbm.at[ri_v.at[k]], qi_rows.at[buf], sems.at[k]),
            pltpu.async_copy(q_hbm.at[rj_v.at[k]], qj_rows.at[buf], sems.at[k]),
        )
        return cps

    inflight = [fire(k) for k in range(NBUF)]

    lane = lax.iota(jnp.int32, L)

    def group(g, carry):
        k, buf = carry
        base = g * L
        pu_vec = jnp.where(uid_v[k, pl.ds(base, L)] >= HALF, D, 0)
        pi_vec = jnp.where(iid_v[k, pl.ds(base, L)] >= HALF, D, 0)
        pj_vec = jnp.where(jid_v[k, pl.ds(base, L)] >= HALF, D, 0)
        res_i = jnp.zeros((L,), jnp.float32)
        res_j = jnp.zeros((L,), jnp.float32)
        for e2 in range(L):
            e = base + e2
            pu = pu_vec[e2]
            pi = pi_vec[e2]
            pj = pj_vec[e2]
            ai = jnp.zeros((L,), jnp.float32)
            aj = jnp.zeros((L,), jnp.float32)
            for c in range(D // L):
                u = u_rows[buf, e, pl.ds(pu + c * L, L)]
                qi = qi_rows[buf, e, pl.ds(pi + c * L, L)]
                qj = qj_rows[buf, e, pl.ds(pj + c * L, L)]
                ai = ai + u * qi
                aj = aj + u * qj
            here = lane == e2
            res_i = jnp.where(here, jnp.broadcast_to(jnp.sum(ai), (L,)), res_i)
            res_j = jnp.where(here, jnp.broadcast_to(jnp.sum(aj), (L,)), res_j)
        out_i[pl.ds(k * CHUNK + base, L)] = res_i
        out_j[pl.ds(k * CHUNK + base, L)] = res_j
        return carry

    for k in range(NCHUNK):
        for cp in inflight[k % NBUF]:
            cp.wait()
        lax.fori_loop(0, CHUNK // L, group, (k, k % NBUF))
        if k + NBUF < NCHUNK:
            inflight[k % NBUF] = fire(k + NBUF)

    pltpu.sync_copy(out_i, predi_hbm.at[pl.ds(wid * BPW, BPW)])
    pltpu.sync_copy(out_j, predj_hbm.at[pl.ds(wid * BPW, BPW)])


HALF = 50048
_TBW = 23 * 128


def _repack_body(a_ref, b_ref, out_ref):
    left = jnp.swapaxes(a_ref[...], 0, 1)
    right = jnp.swapaxes(b_ref[...], 0, 1)
    out_ref[...] = jnp.concatenate([left, right], axis=1)


def _repack(table):
    xt = jnp.swapaxes(table, 0, 1)
    return pl.pallas_call(
        _repack_body,
        grid=(HALF // _TBW,),
        in_specs=[
            pl.BlockSpec((D, _TBW), lambda i: (0, i)),
            pl.BlockSpec((D, _TBW), lambda i: (0, HALF // _TBW + i)),
        ],
        out_specs=pl.BlockSpec((_TBW, W), lambda i: (i, 0)),
        out_shape=jax.ShapeDtypeStruct((HALF, W), jnp.float32),
    )(xt, xt)


def kernel(uid, iid, jid, P, Q):
    uid3 = uid.reshape(NW, NCHUNK, CHUNK)
    iid3 = iid.reshape(NW, NCHUNK, CHUNK)
    jid3 = jid.reshape(NW, NCHUNK, CHUNK)
    p2 = _repack(P)
    q2 = _repack(Q)
    assert p2.shape == (HALF, W) and q2.shape == (HALF, W)

    mesh = plsc.VectorSubcoreMesh(core_axis_name="c", subcore_axis_name="s")
    fn = pl.kernel(
        _body,
        out_type=(
            jax.ShapeDtypeStruct((B,), jnp.float32),
            jax.ShapeDtypeStruct((B,), jnp.float32),
        ),
        mesh=mesh,
        compiler_params=pltpu.CompilerParams(
            needs_layout_passes=False, use_tc_tiling_on_sc=True
        ),
        scratch_types=[
            pltpu.VMEM((NCHUNK, CHUNK), jnp.int32),
            pltpu.VMEM((NCHUNK, CHUNK), jnp.int32),
            pltpu.VMEM((NCHUNK, CHUNK), jnp.int32),
            pltpu.VMEM((NCHUNK, CHUNK), jnp.int32),
            pltpu.VMEM((NCHUNK, CHUNK), jnp.int32),
            pltpu.VMEM((NCHUNK, CHUNK), jnp.int32),
            pltpu.VMEM((NBUF, CHUNK, W), jnp.float32),
            pltpu.VMEM((NBUF, CHUNK, W), jnp.float32),
            pltpu.VMEM((NBUF, CHUNK, W), jnp.float32),
            pltpu.VMEM((BPW,), jnp.float32),
            pltpu.VMEM((BPW,), jnp.float32),
            pltpu.SemaphoreType.DMA((NCHUNK,)),
        ],
    )
    predi, predj = fn(uid3, iid3, jid3, p2, q2)
    return predi, predj
